# baseline (speedup 1.0000x reference)
import jax
import jax.numpy as jnp
from jax import lax
from jax.experimental import pallas as pl
from jax.experimental.pallas import tpu as pltpu
from jax.experimental.pallas import tpu_sc as plsc

_VOCAB = 28
_N_ALPHA = 20
_ALPHA_OFFSET = 3
_BLOSUM_DIM = 24

_NC = 2
_NS = 16
_NW = _NC * _NS
_LANES = 16


def _sc_gather_body(src_hbm, table_hbm, out_hbm, idx_v, table_v, col_v, sem):
    del sem
    ntok = idx_v.shape[0]
    wid = lax.axis_index("s") * _NC + lax.axis_index("c")
    base = wid * ntok
    pltpu.sync_copy(table_hbm, table_v.at[pl.ds(0, _VOCAB * _BLOSUM_DIM)])
    pltpu.sync_copy(src_hbm.at[pl.ds(base, ntok)], idx_v)

    def clamp(i, carry):
        v = idx_v[pl.ds(i * _LANES, _LANES)]
        valid = (v >= _ALPHA_OFFSET) & (v < _ALPHA_OFFSET + _N_ALPHA)
        row = jnp.where(valid, v, _VOCAB - 1)
        idx_v[pl.ds(i * _LANES, _LANES)] = row * _BLOSUM_DIM
        return carry

    lax.fori_loop(0, ntok // _LANES, clamp, 0)

    def gather(i, carry):
        off = idx_v[pl.ds(i * _LANES, _LANES)]
        for j in range(_BLOSUM_DIM):
            col_v[pl.ds(j * ntok + i * _LANES, _LANES)] = plsc.load_gather(
                table_v, [off + j]
            )
        return carry

    lax.fori_loop(0, ntok // _LANES, gather, 0)
    for j in range(_BLOSUM_DIM):
        pltpu.sync_copy(
            col_v.at[pl.ds(j * ntok, ntok)],
            out_hbm.at[j, pl.ds(base, ntok)],
        )


def _sc_gather(srcf, tablef):
    n = srcf.shape[0]
    ntok = n // _NW
    mesh = plsc.VectorSubcoreMesh(core_axis_name="c", subcore_axis_name="s")
    f = pl.kernel(
        _sc_gather_body,
        out_type=jax.ShapeDtypeStruct((_BLOSUM_DIM, n), jnp.float32),
        mesh=mesh,
        compiler_params=pltpu.CompilerParams(needs_layout_passes=False),
        scratch_types=[
            pltpu.VMEM((ntok,), jnp.int32),
            pltpu.VMEM((1024,), jnp.float32),
            pltpu.VMEM((_BLOSUM_DIM * ntok,), jnp.float32),
            pltpu.SemaphoreType.DMA,
        ],
    )
    return f(srcf, tablef)


_XCHUNK = 8


def _tc_assemble_body(x_hbm, cod_ref, out_hbm, tbuf, sems, xsem):
    b = pl.program_id(0)
    nb = pl.num_programs(0)
    B, L, D = x_hbm.shape
    bc = B // _XCHUNK

    @pl.when(b == 0)
    def _start_x():
        for i in range(_XCHUNK):
            pltpu.async_copy(
                x_hbm.at[pl.ds(i * bc, bc)],
                out_hbm.at[pl.ds(i * bc, bc), :, pl.ds(0, D)],
                xsem,
            )

    slot = lax.rem(b, 2)

    @pl.when(b >= 2)
    def _drain_prev():
        pltpu.make_async_copy(
            tbuf.at[slot], out_hbm.at[0, :, pl.ds(D, _BLOSUM_DIM)], sems.at[slot]
        ).wait()

    codt = cod_ref[...]
    n = codt.shape[0]
    eye = (
        lax.broadcasted_iota(jnp.int32, (n, n), 0)
        == lax.broadcasted_iota(jnp.int32, (n, n), 1)
    ).astype(jnp.float32)
    tbuf[slot] = lax.dot_general(
        codt, eye, (((0,), (0,)), ((), ())),
        preferred_element_type=jnp.float32,
    )
    pltpu.async_copy(
        tbuf.at[slot], out_hbm.at[b, :, pl.ds(D, _BLOSUM_DIM)], sems.at[slot]
    )

    @pl.when(b == nb - 1)
    def _drain_all():
        pltpu.make_async_copy(
            tbuf.at[slot], out_hbm.at[0, :, pl.ds(D, _BLOSUM_DIM)], sems.at[slot]
        ).wait()
        pltpu.make_async_copy(
            tbuf.at[1 - slot],
            out_hbm.at[0, :, pl.ds(D, _BLOSUM_DIM)],
            sems.at[1 - slot],
        ).wait()
        for i in range(_XCHUNK):
            pltpu.make_async_copy(
                x_hbm.at[pl.ds(i * bc, bc)],
                out_hbm.at[pl.ds(i * bc, bc), :, pl.ds(0, D)],
                xsem,
            ).wait()


def kernel(src, x, blosum):
    B, L, D = x.shape
    tablef = blosum.reshape(_VOCAB * _BLOSUM_DIM)
    srcf = src.astype(jnp.int32).reshape(B * L)
    coding = _sc_gather(srcf, tablef)
    out = pl.pallas_call(
        _tc_assemble_body,
        grid=(B,),
        in_specs=[
            pl.BlockSpec(memory_space=pltpu.HBM),
            pl.BlockSpec((_BLOSUM_DIM, L), lambda b: (0, b)),
        ],
        out_specs=pl.BlockSpec(memory_space=pltpu.HBM),
        out_shape=jax.ShapeDtypeStruct((B, L, D + _BLOSUM_DIM), jnp.float32),
        scratch_shapes=[
            pltpu.VMEM((2, L, _BLOSUM_DIM), jnp.float32),
            pltpu.SemaphoreType.DMA((2,)),
            pltpu.SemaphoreType.DMA,
        ],
    )(x, coding)
    return out


# speedup vs baseline: 12.7305x; 12.7305x over previous
import jax
import jax.numpy as jnp
from jax import lax
from jax.experimental import pallas as pl
from jax.experimental.pallas import tpu as pltpu
from jax.experimental.pallas import tpu_sc as plsc

_VOCAB = 28
_N_ALPHA = 20
_ALPHA_OFFSET = 3
_BLOSUM_DIM = 24

_NC = 2
_NS = 16
_NW = _NC * _NS
_LANES = 16


def _sc_gather_body(src_hbm, table_hbm, out_hbm, idx_v, table_v, col_v, sem):
    del sem
    ntok = idx_v.shape[0]
    wid = lax.axis_index("s") * _NC + lax.axis_index("c")
    base = wid * ntok
    pltpu.sync_copy(table_hbm, table_v.at[pl.ds(0, _VOCAB * _BLOSUM_DIM)])
    pltpu.sync_copy(src_hbm.at[pl.ds(base, ntok)], idx_v)

    def clamp(i, carry):
        v = idx_v[pl.ds(i * _LANES, _LANES)]
        valid = (v >= _ALPHA_OFFSET) & (v < _ALPHA_OFFSET + _N_ALPHA)
        row = jnp.where(valid, v, _VOCAB - 1)
        idx_v[pl.ds(i * _LANES, _LANES)] = row * _BLOSUM_DIM
        return carry

    lax.fori_loop(0, ntok // _LANES, clamp, 0)

    def gather(i, carry):
        off = idx_v[pl.ds(i * _LANES, _LANES)]
        for j in range(_BLOSUM_DIM):
            col_v[pl.ds(j * ntok + i * _LANES, _LANES)] = plsc.load_gather(
                table_v, [off + j]
            )
        return carry

    lax.fori_loop(0, ntok // _LANES, gather, 0)
    for j in range(_BLOSUM_DIM):
        pltpu.sync_copy(
            col_v.at[pl.ds(j * ntok, ntok)],
            out_hbm.at[j, pl.ds(base, ntok)],
        )


def _sc_gather(srcf, tablef):
    n = srcf.shape[0]
    ntok = n // _NW
    mesh = plsc.VectorSubcoreMesh(core_axis_name="c", subcore_axis_name="s")
    f = pl.kernel(
        _sc_gather_body,
        out_type=jax.ShapeDtypeStruct((_BLOSUM_DIM, n), jnp.float32),
        mesh=mesh,
        compiler_params=pltpu.CompilerParams(needs_layout_passes=False),
        scratch_types=[
            pltpu.VMEM((ntok,), jnp.int32),
            pltpu.VMEM((1024,), jnp.float32),
            pltpu.VMEM((_BLOSUM_DIM * ntok,), jnp.float32),
            pltpu.SemaphoreType.DMA,
        ],
    )
    return f(srcf, tablef)


def _tc_concat_body(x_ref, cod_ref, out_ref):
    codt = cod_ref[...]
    n = codt.shape[0]
    eye = (
        lax.broadcasted_iota(jnp.int32, (n, n), 0)
        == lax.broadcasted_iota(jnp.int32, (n, n), 1)
    ).astype(jnp.float32)
    cod = lax.dot_general(
        codt, eye, (((0,), (0,)), ((), ())),
        preferred_element_type=jnp.float32,
    )
    out_ref[0] = jnp.concatenate([x_ref[0], cod], axis=1)


def kernel(src, x, blosum):
    B, L, D = x.shape
    tablef = blosum.reshape(_VOCAB * _BLOSUM_DIM)
    srcf = src.astype(jnp.int32).reshape(B * L)
    coding = _sc_gather(srcf, tablef)
    LS = 2
    out = pl.pallas_call(
        _tc_concat_body,
        grid=(B, LS),
        in_specs=[
            pl.BlockSpec((1, L // LS, D), lambda b, l: (b, l, 0)),
            pl.BlockSpec((_BLOSUM_DIM, L // LS), lambda b, l: (0, b * LS + l)),
        ],
        out_specs=pl.BlockSpec(
            (1, L // LS, D + _BLOSUM_DIM), lambda b, l: (b, l, 0)
        ),
        out_shape=jax.ShapeDtypeStruct((B, L, D + _BLOSUM_DIM), jnp.float32),
    )(x, coding)
    return out


# speedup vs baseline: 14.8349x; 1.1653x over previous
import jax
import jax.numpy as jnp
from jax import lax
from jax.experimental import pallas as pl
from jax.experimental.pallas import tpu as pltpu
from jax.experimental.pallas import tpu_sc as plsc

_VOCAB = 28
_N_ALPHA = 20
_ALPHA_OFFSET = 3
_BLOSUM_DIM = 24

_NC = 2
_NS = 16
_NW = _NC * _NS
_LANES = 16


def _sc_gather_body(src_hbm, table_hbm, out_hbm, idx_v, table_v, col_v, sem):
    del sem
    ntok = idx_v.shape[0]
    wid = lax.axis_index("s") * _NC + lax.axis_index("c")
    base = wid * ntok
    pltpu.sync_copy(table_hbm, table_v.at[pl.ds(0, _VOCAB * _BLOSUM_DIM)])
    pltpu.sync_copy(src_hbm.at[pl.ds(base, ntok)], idx_v)

    def clamp(i, carry):
        v = idx_v[pl.ds(i * _LANES, _LANES)]
        valid = (v >= _ALPHA_OFFSET) & (v < _ALPHA_OFFSET + _N_ALPHA)
        row = jnp.where(valid, v, _VOCAB - 1)
        idx_v[pl.ds(i * _LANES, _LANES)] = row * _BLOSUM_DIM
        return carry

    lax.fori_loop(0, ntok // _LANES, clamp, 0)

    def gather(i, carry):
        off = idx_v[pl.ds(i * _LANES, _LANES)]
        for j in range(_BLOSUM_DIM):
            col_v[pl.ds(j * ntok + i * _LANES, _LANES)] = plsc.load_gather(
                table_v, [off + j]
            )
        return carry

    lax.fori_loop(0, ntok // _LANES, gather, 0)
    for j in range(_BLOSUM_DIM):
        pltpu.sync_copy(
            col_v.at[pl.ds(j * ntok, ntok)],
            out_hbm.at[j, pl.ds(base, ntok)],
        )


def _sc_gather(srcf, tablef):
    n = srcf.shape[0]
    ntok = n // _NW
    mesh = plsc.VectorSubcoreMesh(core_axis_name="c", subcore_axis_name="s")
    f = pl.kernel(
        _sc_gather_body,
        out_type=jax.ShapeDtypeStruct((_BLOSUM_DIM, n), jnp.float32),
        mesh=mesh,
        compiler_params=pltpu.CompilerParams(needs_layout_passes=False),
        scratch_types=[
            pltpu.VMEM((ntok,), jnp.int32),
            pltpu.VMEM((1024,), jnp.float32),
            pltpu.VMEM((_BLOSUM_DIM * ntok,), jnp.float32),
            pltpu.SemaphoreType.DMA,
        ],
    )
    return f(srcf, tablef)


def _tc_concat_body(x_ref, cod_ref, out_ref):
    bm, ln, d = x_ref.shape
    codt = cod_ref[...]
    n = codt.shape[0]
    eye = (
        lax.broadcasted_iota(jnp.int32, (n, n), 0)
        == lax.broadcasted_iota(jnp.int32, (n, n), 1)
    ).astype(jnp.float32)
    for i in range(bm):
        cod = lax.dot_general(
            codt[:, i * ln:(i + 1) * ln], eye, (((0,), (0,)), ((), ())),
            preferred_element_type=jnp.float32,
        )
        out_ref[i] = jnp.concatenate([x_ref[i], cod], axis=1)


def kernel(src, x, blosum):
    B, L, D = x.shape
    tablef = blosum.reshape(_VOCAB * _BLOSUM_DIM)
    srcf = src.astype(jnp.int32).reshape(B * L)
    coding = _sc_gather(srcf, tablef)
    BM = 2
    out = pl.pallas_call(
        _tc_concat_body,
        grid=(B // BM,),
        in_specs=[
            pl.BlockSpec((BM, L, D), lambda b: (b, 0, 0)),
            pl.BlockSpec((_BLOSUM_DIM, BM * L), lambda b: (0, b)),
        ],
        out_specs=pl.BlockSpec((BM, L, D + _BLOSUM_DIM), lambda b: (b, 0, 0)),
        out_shape=jax.ShapeDtypeStruct((B, L, D + _BLOSUM_DIM), jnp.float32),
    )(x, coding)
    return out


# speedup vs baseline: 14.9447x; 1.0074x over previous
import jax
import jax.numpy as jnp
from jax import lax
from jax.experimental import pallas as pl
from jax.experimental.pallas import tpu as pltpu
from jax.experimental.pallas import tpu_sc as plsc

_VOCAB = 28
_N_ALPHA = 20
_ALPHA_OFFSET = 3
_BLOSUM_DIM = 24

_NC = 2
_NS = 16
_NW = _NC * _NS
_LANES = 16


def _sc_gather_body(src_hbm, table_hbm, out_hbm, idx_v, table_v, col_v, sem):
    del sem
    ntok = idx_v.shape[0]
    wid = lax.axis_index("s") * _NC + lax.axis_index("c")
    base = wid * ntok
    pltpu.sync_copy(table_hbm, table_v.at[pl.ds(0, _VOCAB * _BLOSUM_DIM)])
    pltpu.sync_copy(src_hbm.at[pl.ds(base, ntok)], idx_v)

    def clamp(i, carry):
        v = idx_v[pl.ds(i * _LANES, _LANES)]
        valid = (v >= _ALPHA_OFFSET) & (v < _ALPHA_OFFSET + _N_ALPHA)
        row = jnp.where(valid, v, _VOCAB - 1)
        idx_v[pl.ds(i * _LANES, _LANES)] = row * _BLOSUM_DIM
        return carry

    lax.fori_loop(0, ntok // _LANES, clamp, 0)

    def gather(i, carry):
        off = idx_v[pl.ds(i * _LANES, _LANES)]
        for j in range(_BLOSUM_DIM):
            col_v[pl.ds(j * ntok + i * _LANES, _LANES)] = plsc.load_gather(
                table_v, [off + j]
            )
        return carry

    lax.fori_loop(0, ntok // _LANES, gather, 0)
    for j in range(_BLOSUM_DIM):
        pltpu.sync_copy(
            col_v.at[pl.ds(j * ntok, ntok)],
            out_hbm.at[j, pl.ds(base, ntok)],
        )


def _sc_gather(srcf, tablef):
    n = srcf.shape[0]
    ntok = n // _NW
    mesh = plsc.VectorSubcoreMesh(core_axis_name="c", subcore_axis_name="s")
    f = pl.kernel(
        _sc_gather_body,
        out_type=jax.ShapeDtypeStruct((_BLOSUM_DIM, n), jnp.float32),
        mesh=mesh,
        compiler_params=pltpu.CompilerParams(needs_layout_passes=False),
        scratch_types=[
            pltpu.VMEM((ntok,), jnp.int32),
            pltpu.VMEM((1024,), jnp.float32),
            pltpu.VMEM((_BLOSUM_DIM * ntok,), jnp.float32),
            pltpu.SemaphoreType.DMA,
        ],
    )
    return f(srcf, tablef)


def _tc_concat_body(x_ref, cod_ref, out_ref):
    bm, ln, d = x_ref.shape
    codt = cod_ref[...]
    n = codt.shape[0]
    eye = (
        lax.broadcasted_iota(jnp.int32, (n, n), 0)
        == lax.broadcasted_iota(jnp.int32, (n, n), 1)
    ).astype(jnp.float32)
    for i in range(bm):
        cod = lax.dot_general(
            codt[:, i * ln:(i + 1) * ln], eye, (((0,), (0,)), ((), ())),
            preferred_element_type=jnp.float32,
        )
        out_ref[i] = jnp.concatenate([x_ref[i], cod], axis=1)


def kernel(src, x, blosum):
    B, L, D = x.shape
    tablef = blosum.reshape(_VOCAB * _BLOSUM_DIM)
    srcf = src.astype(jnp.int32).reshape(B * L)
    coding = _sc_gather(srcf, tablef)
    BM = 4
    out = pl.pallas_call(
        _tc_concat_body,
        grid=(B // BM,),
        in_specs=[
            pl.BlockSpec((BM, L, D), lambda b: (b, 0, 0)),
            pl.BlockSpec((_BLOSUM_DIM, BM * L), lambda b: (0, b)),
        ],
        out_specs=pl.BlockSpec((BM, L, D + _BLOSUM_DIM), lambda b: (b, 0, 0)),
        out_shape=jax.ShapeDtypeStruct((B, L, D + _BLOSUM_DIM), jnp.float32),
    )(x, coding)
    return out


# speedup vs baseline: 14.9724x; 1.0019x over previous
import jax
import jax.numpy as jnp
from jax import lax
from jax.experimental import pallas as pl
from jax.experimental.pallas import tpu as pltpu
from jax.experimental.pallas import tpu_sc as plsc

_VOCAB = 28
_N_ALPHA = 20
_ALPHA_OFFSET = 3
_BLOSUM_DIM = 24

_NC = 2
_NS = 16
_NW = _NC * _NS
_LANES = 16


def _sc_gather_body(src_hbm, table_hbm, out_hbm, idx_v, table_v, col_v, sem):
    del sem
    ntok = idx_v.shape[0]
    wid = lax.axis_index("s") * _NC + lax.axis_index("c")
    base = wid * ntok
    pltpu.sync_copy(table_hbm, table_v.at[pl.ds(0, _VOCAB * _BLOSUM_DIM)])
    pltpu.sync_copy(src_hbm.at[pl.ds(base, ntok)], idx_v)

    def gather(i, carry):
        v = idx_v[pl.ds(i * _LANES, _LANES)]
        valid = (v >= _ALPHA_OFFSET) & (v < _ALPHA_OFFSET + _N_ALPHA)
        off = jnp.where(valid, v, _VOCAB - 1) * _BLOSUM_DIM
        for j in range(_BLOSUM_DIM):
            col_v[pl.ds(j * ntok + i * _LANES, _LANES)] = plsc.load_gather(
                table_v, [off + j]
            )
        return carry

    lax.fori_loop(0, ntok // _LANES, gather, 0)

    for j in range(_BLOSUM_DIM):
        pltpu.sync_copy(
            col_v.at[pl.ds(j * ntok, ntok)],
            out_hbm.at[j, pl.ds(base, ntok)],
        )


def _sc_gather(srcf, tablef):
    n = srcf.shape[0]
    ntok = n // _NW
    mesh = plsc.VectorSubcoreMesh(core_axis_name="c", subcore_axis_name="s")
    f = pl.kernel(
        _sc_gather_body,
        out_type=jax.ShapeDtypeStruct((_BLOSUM_DIM, n), jnp.float32),
        mesh=mesh,
        compiler_params=pltpu.CompilerParams(needs_layout_passes=False),
        scratch_types=[
            pltpu.VMEM((ntok,), jnp.int32),
            pltpu.VMEM((1024,), jnp.float32),
            pltpu.VMEM((_BLOSUM_DIM * ntok,), jnp.float32),
            pltpu.SemaphoreType.DMA,
        ],
    )
    return f(srcf, tablef)


def _tc_concat_body(x_ref, cod_ref, out_ref):
    bm, ln, d = x_ref.shape
    codt = cod_ref[...]
    n = codt.shape[0]
    eye = (
        lax.broadcasted_iota(jnp.int32, (n, n), 0)
        == lax.broadcasted_iota(jnp.int32, (n, n), 1)
    ).astype(jnp.float32)
    for i in range(bm):
        cod = lax.dot_general(
            codt[:, i * ln:(i + 1) * ln], eye, (((0,), (0,)), ((), ())),
            preferred_element_type=jnp.float32,
        )
        out_ref[i] = jnp.concatenate([x_ref[i], cod], axis=1)


def kernel(src, x, blosum):
    B, L, D = x.shape
    tablef = blosum.reshape(_VOCAB * _BLOSUM_DIM)
    srcf = src.astype(jnp.int32).reshape(B * L)
    coding = _sc_gather(srcf, tablef)
    BM = 4
    out = pl.pallas_call(
        _tc_concat_body,
        grid=(B // BM,),
        in_specs=[
            pl.BlockSpec((BM, L, D), lambda b: (b, 0, 0)),
            pl.BlockSpec((_BLOSUM_DIM, BM * L), lambda b: (0, b)),
        ],
        out_specs=pl.BlockSpec((BM, L, D + _BLOSUM_DIM), lambda b: (b, 0, 0)),
        out_shape=jax.ShapeDtypeStruct((B, L, D + _BLOSUM_DIM), jnp.float32),
    )(x, coding)
    return out
